# baseline (speedup 1.0000x reference)
import jax
import jax.numpy as jnp
from jax.experimental import pallas as pl
from jax.experimental.pallas import tpu as pltpu

_S = 4
_C = 16
_H = 32
_OC = 16
_CIN = _C + 4


def _agg_kernel(x_ref, wd_ref, bv_ref, wgb_ref, wvm_ref, bg_ref, wa_ref,
                ba_ref, wf_ref, bf_ref, out_ref):
    f32 = jnp.float32
    T = x_ref.shape[2]
    S, C, H, OC = _S, _C, _H, _OC

    featall = jnp.concatenate([x_ref[s, 0:C] for s in range(S)], axis=0)
    dall = jnp.concatenate([x_ref[s, C:_CIN] for s in range(S)], axis=0)

    vall = jnp.dot(wd_ref[...], dall, preferred_element_type=f32)
    vall = jnp.maximum(vall + jnp.broadcast_to(bv_ref[...], (S * C, T)), 0.0)
    img = featall + vall

    mean = (img[0:C] + img[C:2 * C] + img[2 * C:3 * C] + img[3 * C:4 * C]) * (1.0 / S)
    mean4 = jnp.concatenate([mean] * S, axis=0)
    dlt = img - mean4
    sq = dlt * dlt
    var = (sq[0:C] + sq[C:2 * C] + sq[2 * C:3 * C] + sq[3 * C:4 * C]) * (1.0 / (S - 1))

    vm = jnp.concatenate([var, mean], axis=0)
    gvm = (jnp.dot(wvm_ref[...], vm, preferred_element_type=f32)
           + jnp.broadcast_to(bg_ref[...], (H, T)))
    gvm4 = jnp.concatenate([gvm] * S, axis=0)
    gf = jnp.maximum(jnp.dot(wgb_ref[...], img, preferred_element_type=f32)
                     + gvm4, 0.0)

    p = gf * jnp.broadcast_to(wa_ref[...], (S * H, T))
    ba_b = jnp.broadcast_to(ba_ref[...], (1, T))
    scores = jnp.concatenate(
        [jnp.maximum(jnp.sum(p[H * s:H * s + H], axis=0, keepdims=True) + ba_b,
                     0.0) for s in range(S)], axis=0)

    m = jnp.max(scores, axis=0, keepdims=True)
    e = jnp.exp(scores - m)
    w = e * pl.reciprocal(jnp.sum(e, axis=0, keepdims=True), approx=False)
    acc = w[0:1] * gf[0:H]
    for s in range(1, S):
        acc = acc + w[s:s + 1] * gf[H * s:H * s + H]

    out = jnp.dot(wf_ref[...], acc, preferred_element_type=f32)
    out = out + jnp.broadcast_to(bf_ref[...], (OC, T))
    out_ref[...] = jnp.maximum(out, 0.0).reshape(1, OC, T).astype(out_ref.dtype)


def kernel(x, wv, bv, wg, bg, wa, ba, wf, bf, *, tile_n=16384):
    B, P, S, Cin = x.shape
    C = Cin - 4
    H = wg.shape[1]
    OC = wf.shape[1]
    N = B * P
    f32 = jnp.float32

    tile = next((t for t in (tile_n, 8192, 4096, 1024, 512, 256, 128) if P % t == 0), P)
    jb = P // tile

    xt = jnp.transpose(x.reshape(N, S, Cin), (1, 2, 0))

    wd = jnp.zeros((S * C, S * 4), f32)
    wgb = jnp.zeros((S * H, S * C), f32)
    for s in range(S):
        wd = wd.at[s * C:(s + 1) * C, s * 4:(s + 1) * 4].set(wv.T)
        wgb = wgb.at[s * H:(s + 1) * H, s * C:(s + 1) * C].set(wg[:C].T)
    bv4 = jnp.tile(bv.reshape(C, 1), (S, 1))
    wvm = jnp.concatenate([wg[C:2 * C].T, wg[2 * C:3 * C].T], axis=1)
    bg_c = bg.reshape(H, 1)
    wa4 = jnp.tile(wa.reshape(H, 1), (S, 1))
    ba_c = ba.reshape(1, 1)
    wfT = wf.T
    bf_c = bf.reshape(OC, 1)

    def full(a):
        nd = a.ndim
        return pl.BlockSpec(a.shape, lambda b, j, _nd=nd: (0,) * _nd)

    out = pl.pallas_call(
        _agg_kernel,
        out_shape=jax.ShapeDtypeStruct((B, OC, P), f32),
        grid=(B, jb),
        in_specs=[
            pl.BlockSpec((S, Cin, tile), lambda b, j, _jb=jb: (0, 0, b * _jb + j)),
            full(wd), full(bv4), full(wgb), full(wvm), full(bg_c),
            full(wa4), full(ba_c), full(wfT), full(bf_c),
        ],
        out_specs=pl.BlockSpec((1, OC, tile), lambda b, j: (b, 0, j)),
        compiler_params=pltpu.CompilerParams(
            dimension_semantics=("parallel", "parallel"),
            vmem_limit_bytes=64 * 1024 * 1024),
    )(xt, wd, bv4, wgb, wvm, bg_c, wa4, ba_c, wfT, bf_c)

    return jnp.transpose(out, (0, 2, 1))


# speedup vs baseline: 1.5755x; 1.5755x over previous
import jax
import jax.numpy as jnp
from jax.experimental import pallas as pl
from jax.experimental.pallas import tpu as pltpu

_S = 4
_C = 16
_H = 32
_OC = 16
_CIN = _C + 4


def _agg_kernel(x_ref, wd_ref, bv_ref, wgb_ref, wvm_ref, bg_ref, wsc_ref,
                ba_ref, wsel_ref, wf_ref, bf_ref, mavg_ref, msum_ref, out_ref):
    f32 = jnp.float32
    K = x_ref.shape[2]
    T = K * 128
    S, C, H, OC = _S, _C, _H, _OC

    rows = []
    for c in range(_CIN):
        a = x_ref[0, c]
        t = jnp.transpose(a, (1, 0, 2))
        rows.append(t.reshape(S, T))
    featall = jnp.concatenate(rows[:C], axis=0)
    dall = jnp.concatenate(rows[C:], axis=0)

    vall = jnp.dot(wd_ref[...], dall, preferred_element_type=f32)
    vall = jnp.maximum(vall + jnp.broadcast_to(bv_ref[...], (C * S, T)), 0.0)
    img = featall + vall

    mean = jnp.dot(mavg_ref[...], img, preferred_element_type=f32)
    mean4 = jnp.repeat(mean, S, axis=0)
    dlt = img - mean4
    sq = dlt * dlt
    var = jnp.dot(msum_ref[...], sq, preferred_element_type=f32)

    vm = jnp.concatenate([var, mean], axis=0)
    gvm = (jnp.dot(wvm_ref[...], vm, preferred_element_type=f32)
           + jnp.broadcast_to(bg_ref[...], (H, T)))
    gvm4 = jnp.repeat(gvm, S, axis=0)
    gf = jnp.maximum(jnp.dot(wgb_ref[...], img, preferred_element_type=f32)
                     + gvm4, 0.0)

    scores = jnp.dot(wsc_ref[...], gf, preferred_element_type=f32)
    scores = jnp.maximum(scores + jnp.broadcast_to(ba_ref[...], (S, T)), 0.0)

    m = jnp.max(scores, axis=0, keepdims=True)
    e = jnp.exp(scores - m)
    w = e * pl.reciprocal(jnp.sum(e, axis=0, keepdims=True), approx=False)
    wrep = jnp.tile(w, (H, 1))
    acc = jnp.dot(wsel_ref[...], gf * wrep, preferred_element_type=f32)

    out = jnp.dot(wf_ref[...], acc, preferred_element_type=f32)
    out = out + jnp.broadcast_to(bf_ref[...], (OC, T))
    out_ref[...] = jnp.maximum(out, 0.0).reshape(1, OC, T).astype(out_ref.dtype)


def kernel(x, wv, bv, wg, bg, wa, ba, wf, bf, *, kt=32):
    B, P, S, Cin = x.shape
    C = Cin - 4
    H = wg.shape[1]
    OC = wf.shape[1]
    f32 = jnp.float32

    PT = P // 128
    K = next((k for k in (kt, 32, 16, 8, 4, 2, 1) if PT % k == 0), 1)
    jb = PT // K

    x5 = jnp.transpose(x.reshape(B, PT, 128, S, Cin), (0, 4, 1, 3, 2))

    eye = jnp.eye(S, dtype=f32)
    wd = (wv.T[:, None, :, None] * eye[None, :, None, :]).reshape(C * S, 4 * S)
    bv4 = jnp.repeat(bv.reshape(C, 1), S, axis=0)
    wgb = (wg[:C].T[:, None, :, None] * eye[None, :, None, :]).reshape(H * S, C * S)
    wvm = jnp.concatenate([wg[C:2 * C].T, wg[2 * C:3 * C].T], axis=1)
    bg_c = bg.reshape(H, 1)
    wsc = (wa.reshape(1, H, 1) * eye[:, None, :]).reshape(S, H * S)
    ba_c = ba.reshape(1, 1)
    wsel = (jnp.eye(H, dtype=f32)[:, :, None] *
            jnp.ones((1, 1, S), f32)).reshape(H, H * S)
    wfT = wf.T
    bf_c = bf.reshape(OC, 1)
    csel = (jnp.eye(C, dtype=f32)[:, :, None] *
            jnp.ones((1, 1, S), f32)).reshape(C, C * S)
    mavg = csel * (1.0 / S)
    msum = csel * (1.0 / (S - 1))

    def full(a):
        nd = a.ndim
        return pl.BlockSpec(a.shape, lambda b, j, _nd=nd: (0,) * _nd)

    out = pl.pallas_call(
        _agg_kernel,
        out_shape=jax.ShapeDtypeStruct((B, OC, P), f32),
        grid=(B, jb),
        in_specs=[
            pl.BlockSpec((1, Cin, K, S, 128), lambda b, j: (b, 0, j, 0, 0)),
            full(wd), full(bv4), full(wgb), full(wvm), full(bg_c),
            full(wsc), full(ba_c), full(wsel), full(wfT), full(bf_c),
            full(mavg), full(msum),
        ],
        out_specs=pl.BlockSpec((1, OC, K * 128), lambda b, j: (b, 0, j)),
        compiler_params=pltpu.CompilerParams(
            dimension_semantics=("parallel", "parallel"),
            vmem_limit_bytes=64 * 1024 * 1024),
    )(x5, wd, bv4, wgb, wvm, bg_c, wsc, ba_c, wsel, wfT, bf_c, mavg, msum)

    return jnp.transpose(out, (0, 2, 1))


# speedup vs baseline: 1.7953x; 1.1395x over previous
import jax
import jax.numpy as jnp
from jax.experimental import pallas as pl
from jax.experimental.pallas import tpu as pltpu

_S = 4
_C = 16
_H = 32
_OC = 16
_CIN = _C + 4


def _agg_kernel(x_ref, wd_ref, bv_ref, wgb_ref, wvm_ref, bg_ref, wa_ref,
                ba_ref, wf_ref, bf_ref, mavg_ref, msum_ref, out_ref):
    f32 = jnp.float32
    K = x_ref.shape[2]
    T = K * 128
    S, C, H, OC = _S, _C, _H, _OC

    rows = []
    for c in range(_CIN):
        a = x_ref[0, c]
        t = jnp.transpose(a, (1, 0, 2))
        rows.append(t.reshape(S, T))
    featall = jnp.concatenate(rows[:C], axis=0)
    dall = jnp.concatenate(rows[C:], axis=0)

    vall = jnp.dot(wd_ref[...], dall, preferred_element_type=f32)
    vall = jnp.maximum(vall + jnp.broadcast_to(bv_ref[...], (C * S, T)), 0.0)
    img = featall + vall

    mean = jnp.dot(mavg_ref[...], img, preferred_element_type=f32)
    mean4 = jnp.repeat(mean, S, axis=0)
    dlt = img - mean4
    sq = dlt * dlt
    var = jnp.dot(msum_ref[...], sq, preferred_element_type=f32)

    vm = jnp.concatenate([var, mean], axis=0)
    gvm = (jnp.dot(wvm_ref[...], vm, preferred_element_type=f32)
           + jnp.broadcast_to(bg_ref[...], (H, T)))
    gvm4 = jnp.tile(gvm, (S, 1))
    gf = jnp.maximum(jnp.dot(wgb_ref[...], img, preferred_element_type=f32)
                     + gvm4, 0.0)

    p = gf * jnp.broadcast_to(wa_ref[...], (S * H, T))
    ba_b = jnp.broadcast_to(ba_ref[...], (1, T))
    scores = jnp.concatenate(
        [jnp.maximum(jnp.sum(p[H * s:H * s + H], axis=0, keepdims=True) + ba_b,
                     0.0) for s in range(S)], axis=0)

    m = jnp.max(scores, axis=0, keepdims=True)
    e = jnp.exp(scores - m)
    w = e * pl.reciprocal(jnp.sum(e, axis=0, keepdims=True), approx=False)
    acc = w[0:1] * gf[0:H]
    for s in range(1, S):
        acc = acc + w[s:s + 1] * gf[H * s:H * s + H]

    out = jnp.dot(wf_ref[...], acc, preferred_element_type=f32)
    out = out + jnp.broadcast_to(bf_ref[...], (OC, T))
    out_ref[...] = jnp.maximum(out, 0.0).reshape(1, OC, T).astype(out_ref.dtype)


def kernel(x, wv, bv, wg, bg, wa, ba, wf, bf, *, kt=32):
    B, P, S, Cin = x.shape
    C = Cin - 4
    H = wg.shape[1]
    OC = wf.shape[1]
    f32 = jnp.float32

    PT = P // 128
    K = next((k for k in (kt, 32, 16, 8, 4, 2, 1) if PT % k == 0), 1)
    jb = PT // K

    x5 = jnp.transpose(x.reshape(B, PT, 128, S, Cin), (0, 4, 1, 3, 2))

    eye = jnp.eye(S, dtype=f32)
    wd = (wv.T[:, None, :, None] * eye[None, :, None, :]).reshape(C * S, 4 * S)
    bv4 = jnp.repeat(bv.reshape(C, 1), S, axis=0)
    wgb = (wg[:C].T[None, :, :, None] * eye[:, None, None, :]).reshape(S * H, C * S)
    wvm = jnp.concatenate([wg[C:2 * C].T, wg[2 * C:3 * C].T], axis=1)
    bg_c = bg.reshape(H, 1)
    wa4 = jnp.tile(wa.reshape(H, 1), (S, 1))
    ba_c = ba.reshape(1, 1)
    wfT = wf.T
    bf_c = bf.reshape(OC, 1)
    csel = (jnp.eye(C, dtype=f32)[:, :, None] *
            jnp.ones((1, 1, S), f32)).reshape(C, C * S)
    mavg = csel * (1.0 / S)
    msum = csel * (1.0 / (S - 1))

    def full(a):
        nd = a.ndim
        return pl.BlockSpec(a.shape, lambda b, j, _nd=nd: (0,) * _nd)

    out = pl.pallas_call(
        _agg_kernel,
        out_shape=jax.ShapeDtypeStruct((B, OC, P), f32),
        grid=(B, jb),
        in_specs=[
            pl.BlockSpec((1, Cin, K, S, 128), lambda b, j: (b, 0, j, 0, 0)),
            full(wd), full(bv4), full(wgb), full(wvm), full(bg_c),
            full(wa4), full(ba_c), full(wfT), full(bf_c),
            full(mavg), full(msum),
        ],
        out_specs=pl.BlockSpec((1, OC, K * 128), lambda b, j: (b, 0, j)),
        compiler_params=pltpu.CompilerParams(
            dimension_semantics=("parallel", "parallel"),
            vmem_limit_bytes=64 * 1024 * 1024),
    )(x5, wd, bv4, wgb, wvm, bg_c, wa4, ba_c, wfT, bf_c, mavg, msum)

    return jnp.transpose(out, (0, 2, 1))


# speedup vs baseline: 1.9745x; 1.0998x over previous
import jax
import jax.numpy as jnp
from jax.experimental import pallas as pl
from jax.experimental.pallas import tpu as pltpu

_S = 4
_C = 16
_H = 32
_OC = 16
_CIN = _C + 4


def _agg_kernel(x_ref, wd_ref, bv_ref, wgb_ref, wvm_ref, bg_ref, wa_ref,
                ba_ref, wf_ref, bf_ref, mavg_ref, msum_ref, out_ref):
    f32 = jnp.float32
    K = x_ref.shape[2]
    T = K * 128
    S, C, H, OC = _S, _C, _H, _OC

    rows = []
    for c in range(_CIN):
        a = x_ref[0, c]
        t = jnp.transpose(a, (1, 0, 2))
        rows.append(t.reshape(S, T))
    featall = jnp.concatenate(rows[:C], axis=0)
    dall = jnp.concatenate(rows[C:], axis=0)

    vall = jnp.dot(wd_ref[...], dall, preferred_element_type=f32)
    vall = jnp.maximum(vall + jnp.broadcast_to(bv_ref[...], (C * S, T)), 0.0)
    img = featall + vall

    mean = jnp.dot(mavg_ref[...], img, preferred_element_type=f32)
    mean4 = jnp.repeat(mean, S, axis=0)
    dlt = img - mean4
    sq = dlt * dlt
    var = jnp.dot(msum_ref[...], sq, preferred_element_type=f32)

    vm = jnp.concatenate([var, mean], axis=0)
    gvm = (jnp.dot(wvm_ref[...], vm, preferred_element_type=f32)
           + jnp.broadcast_to(bg_ref[...], (H, T)))
    gvm4 = jnp.tile(gvm, (S, 1))
    gf = jnp.maximum(jnp.dot(wgb_ref[...], img, preferred_element_type=f32)
                     + gvm4, 0.0)

    p = gf * jnp.broadcast_to(wa_ref[...], (S * H, T))
    ba_b = jnp.broadcast_to(ba_ref[...], (1, T))
    scores = jnp.concatenate(
        [jnp.maximum(jnp.sum(p[H * s:H * s + H], axis=0, keepdims=True) + ba_b,
                     0.0) for s in range(S)], axis=0)

    m = jnp.max(scores, axis=0, keepdims=True)
    e = jnp.exp(scores - m)
    w = e * pl.reciprocal(jnp.sum(e, axis=0, keepdims=True), approx=False)
    acc = w[0:1] * gf[0:H]
    for s in range(1, S):
        acc = acc + w[s:s + 1] * gf[H * s:H * s + H]

    out = jnp.dot(wf_ref[...], acc, preferred_element_type=f32)
    out = out + jnp.broadcast_to(bf_ref[...], (OC, T))
    out_ref[...] = jnp.maximum(out, 0.0).reshape(1, OC, T).astype(out_ref.dtype)


def kernel(x, wv, bv, wg, bg, wa, ba, wf, bf, *, kt=64):
    B, P, S, Cin = x.shape
    C = Cin - 4
    H = wg.shape[1]
    OC = wf.shape[1]
    f32 = jnp.float32

    PT = P // 128
    K = next((k for k in (kt, 32, 16, 8, 4, 2, 1) if PT % k == 0), 1)
    jb = PT // K

    x5 = jnp.transpose(x.reshape(B, PT, 128, S, Cin), (0, 4, 1, 3, 2))

    eye = jnp.eye(S, dtype=f32)
    wd = (wv.T[:, None, :, None] * eye[None, :, None, :]).reshape(C * S, 4 * S)
    bv4 = jnp.repeat(bv.reshape(C, 1), S, axis=0)
    wgb = (wg[:C].T[None, :, :, None] * eye[:, None, None, :]).reshape(S * H, C * S)
    wvm = jnp.concatenate([wg[C:2 * C].T, wg[2 * C:3 * C].T], axis=1)
    bg_c = bg.reshape(H, 1)
    wa4 = jnp.tile(wa.reshape(H, 1), (S, 1))
    ba_c = ba.reshape(1, 1)
    wfT = wf.T
    bf_c = bf.reshape(OC, 1)
    csel = (jnp.eye(C, dtype=f32)[:, :, None] *
            jnp.ones((1, 1, S), f32)).reshape(C, C * S)
    mavg = csel * (1.0 / S)
    msum = csel * (1.0 / (S - 1))

    def full(a):
        nd = a.ndim
        return pl.BlockSpec(a.shape, lambda b, j, _nd=nd: (0,) * _nd)

    out = pl.pallas_call(
        _agg_kernel,
        out_shape=jax.ShapeDtypeStruct((B, OC, P), f32),
        grid=(B, jb),
        in_specs=[
            pl.BlockSpec((1, Cin, K, S, 128), lambda b, j: (b, 0, j, 0, 0)),
            full(wd), full(bv4), full(wgb), full(wvm), full(bg_c),
            full(wa4), full(ba_c), full(wfT), full(bf_c),
            full(mavg), full(msum),
        ],
        out_specs=pl.BlockSpec((1, OC, K * 128), lambda b, j: (b, 0, j)),
        compiler_params=pltpu.CompilerParams(
            dimension_semantics=("parallel", "parallel"),
            vmem_limit_bytes=64 * 1024 * 1024),
    )(x5, wd, bv4, wgb, wvm, bg_c, wa4, ba_c, wfT, bf_c, mavg, msum)

    return jnp.transpose(out, (0, 2, 1))
